# Initial kernel scaffold; baseline (speedup 1.0000x reference)
import jax
import jax.numpy as jnp
from jax.experimental import pallas as pl


def kernel(x, edge_index, edge_weight, norm, head, G1, G2, B1, B2, r, W):
    raise NotImplementedError("write your pallas kernel here")


# speedup vs baseline: 4.5918x; 4.5918x over previous
import functools

import jax
import jax.numpy as jnp
from jax import lax
from jax.experimental import pallas as pl
from jax.experimental.pallas import tpu as pltpu
import jax.experimental.pallas.tpu_sc as plsc

N = 10000
E = 320000
F = 128

NC = 2
NS = 16
NW = NC * NS

CH = 128
E_PAD = 327680
EPW = E_PAD // NW
NCHUNK = EPW // CH
RPT = 624
REM = N - NS * RPT


def _spmm_body(x_hbm, src_hbm, dst_hbm, w_hbm, zeros_hbm, out_hbm,
               src_v, dst_v, w_v, rows_v, acc, sem):
  c = lax.axis_index("c")
  s = lax.axis_index("s")
  wid = c * NS + s
  ebase = wid * EPW
  rbase = s * RPT

  pltpu.sync_copy(zeros_hbm.at[pl.ds(rbase, RPT)], acc.at[pl.ds(rbase, RPT)])

  @pl.when(s == NS - 1)
  def _():
    pltpu.sync_copy(zeros_hbm.at[pl.ds(NS * RPT, REM)],
                    acc.at[pl.ds(NS * RPT, REM)])

  plsc.subcore_barrier()

  def chunk_body(i, carry):
    off = ebase + i * CH
    pltpu.sync_copy(src_hbm.at[pl.ds(off, CH)], src_v)
    pltpu.sync_copy(dst_hbm.at[pl.ds(off, CH)], dst_v)
    pltpu.sync_copy(w_hbm.at[pl.ds(off, CH)], w_v)
    pltpu.async_copy(x_hbm.at[src_v], rows_v, sem).wait()

    dnums = lax.GatherDimensionNumbers(
        offset_dims=(), collapsed_slice_dims=(0,), start_index_map=(0,))

    def group_body(g, carry2):
      wv = w_v[pl.ds(g * 16, 16)]
      for e in range(16):
        idx = jnp.full((16, 1), e, jnp.int32)
        wb = lax.gather(wv, idx, dnums, (1,),
                        mode=lax.GatherScatterMode.PROMISE_IN_BOUNDS)
        row = g * 16 + e
        for j in range(F // 16):
          sl = pl.ds(j * 16, 16)
          rows_v[row, sl] = rows_v[row, sl] * wb
      return carry2

    lax.fori_loop(0, CH // 16, group_body, 0)

    pltpu.sync_copy(rows_v, acc.at[dst_v], add=True)
    return carry

  lax.fori_loop(0, NCHUNK, chunk_body, 0)
  plsc.subcore_barrier()

  pltpu.sync_copy(acc.at[pl.ds(rbase, RPT)], out_hbm.at[c, pl.ds(rbase, RPT)])

  @pl.when(s == NS - 1)
  def _():
    pltpu.sync_copy(acc.at[pl.ds(NS * RPT, REM)],
                    out_hbm.at[c, pl.ds(NS * RPT, REM)])


@jax.jit
def _spmm_sc(x, src, dst, w, zeros):
  mesh = plsc.VectorSubcoreMesh(core_axis_name="c", subcore_axis_name="s")
  return pl.kernel(
      _spmm_body,
      out_type=jax.ShapeDtypeStruct((NC, N, F), jnp.float32),
      mesh=mesh,
      scratch_types=[
          pltpu.VMEM((CH,), jnp.int32),
          pltpu.VMEM((CH,), jnp.int32),
          pltpu.VMEM((CH,), jnp.float32),
          pltpu.VMEM((CH, F), jnp.float32),
          pltpu.VMEM_SHARED((N, F), jnp.float32),
          pltpu.SemaphoreType.DMA,
      ],
  )(x, src, dst, w, zeros)


def _dense_body(x_ref, nb0_ref, nb1_ref, norm_ref, sel_ref,
                g1_ref, g2_ref, b1_ref, b2_ref, r_ref, w_ref,
                hk_ref, m_ref):
  x = x_ref[...]
  nb = nb0_ref[...] + nb1_ref[...]
  ga = jnp.dot(x, g1_ref[...], preferred_element_type=jnp.float32)
  ga += jnp.dot(nb, g2_ref[...], preferred_element_type=jnp.float32)
  gamma = jnp.where(ga >= 0, ga, 0.2 * ga) + 1.0
  be = jnp.dot(x, b1_ref[...], preferred_element_type=jnp.float32)
  be += jnp.dot(nb, b2_ref[...], preferred_element_type=jnp.float32)
  beta = jnp.where(be >= 0, be, 0.2 * be)
  m = x + gamma * r_ref[...] + beta - nb
  m_ref[...] = m
  t = x + nb + sel_ref[0, 0] * m
  hk = jnp.dot(t, w_ref[...], preferred_element_type=jnp.float32)
  hk_ref[...] = hk / (norm_ref[...] + 1.0)


BLK = 1000


@jax.jit
def _dense_tc(x, nb0, nb1, norm, sel, G1, G2, B1, B2, r, W):
  grid = (N // BLK,)
  row_spec = pl.BlockSpec((BLK, F), lambda i: (i, 0))
  full_spec = pl.BlockSpec((F, F), lambda i: (0, 0))
  return pl.pallas_call(
      _dense_body,
      grid=grid,
      in_specs=[
          row_spec, row_spec, row_spec,
          pl.BlockSpec((BLK, 1), lambda i: (i, 0)),
          pl.BlockSpec(memory_space=pltpu.SMEM),
          full_spec, full_spec, full_spec, full_spec,
          pl.BlockSpec((1, F), lambda i: (0, 0)),
          full_spec,
      ],
      out_specs=[row_spec, row_spec],
      out_shape=[
          jax.ShapeDtypeStruct((N, F), jnp.float32),
          jax.ShapeDtypeStruct((N, F), jnp.float32),
      ],
  )(x, nb0, nb1, norm, sel, G1, G2, B1, B2, r, W)


def kernel(x, edge_index, edge_weight, norm, head, G1, G2, B1, B2, r, W):
  src = edge_index[0]
  dst = edge_index[1]
  pad = E_PAD - E
  src_p = jnp.concatenate([src, jnp.zeros((pad,), jnp.int32)])
  dst_p = jnp.concatenate([dst, jnp.zeros((pad,), jnp.int32)])
  w_p = jnp.concatenate([edge_weight, jnp.zeros((pad,), jnp.float32)])
  zeros = jnp.zeros((N, F), jnp.float32)
  partial = _spmm_sc(x, src_p, dst_p, w_p, zeros)
  sel = jnp.where(jnp.asarray(head) != 0, 0.0, 1.0).astype(jnp.float32)
  sel = jnp.reshape(sel, (1, 1))
  hk, m = _dense_tc(x, partial[0], partial[1], norm, sel, G1, G2, B1, B2, r, W)
  return (hk, m)


# speedup vs baseline: 6.2334x; 1.3575x over previous
import functools

import jax
import jax.numpy as jnp
from jax import lax
from jax.experimental import pallas as pl
from jax.experimental.pallas import tpu as pltpu
import jax.experimental.pallas.tpu_sc as plsc

N = 10000
E = 320000
F = 128

NC = 2
NS = 16
NW = NC * NS

CH = 128
E_PAD = 327680
EPW = E_PAD // NW
NCHUNK = EPW // CH
RPT = 624
REM = N - NS * RPT


_DNUMS = lax.GatherDimensionNumbers(
    offset_dims=(), collapsed_slice_dims=(0,), start_index_map=(0,))


def _scale_rows(rows_v, w_all, i):
  def group_body(g, carry):
    wv = w_all[pl.ds(i * CH + g * 16, 16)]
    for e in range(16):
      idx = jnp.full((16, 1), e, jnp.int32)
      wb = lax.gather(wv, idx, _DNUMS, (1,),
                      mode=lax.GatherScatterMode.PROMISE_IN_BOUNDS)
      row = g * 16 + e
      for j in range(F // 16):
        sl = pl.ds(j * 16, 16)
        rows_v[row, sl] = rows_v[row, sl] * wb
    return carry

  lax.fori_loop(0, CH // 16, group_body, 0)


CPH = NCHUNK // 2


def _spmm_body(x_hbm, src_hbm, dst_hbm, w_hbm, zeros_hbm, out_hbm,
               src_half, dst_half, w_half, rows0, rows1, acc,
               gsem0, gsem1):
  c = lax.axis_index("c")
  s = lax.axis_index("s")
  wid = c * NS + s
  rbase = s * RPT

  pltpu.sync_copy(zeros_hbm.at[pl.ds(rbase, RPT)], acc.at[pl.ds(rbase, RPT)])

  @pl.when(s == NS - 1)
  def _():
    pltpu.sync_copy(zeros_hbm.at[pl.ds(NS * RPT, REM)],
                    acc.at[pl.ds(NS * RPT, REM)])

  plsc.subcore_barrier()

  for h in range(2):
    pltpu.sync_copy(src_hbm.at[wid, pl.ds(h * CPH, CPH)], src_half)
    pltpu.sync_copy(dst_hbm.at[wid, pl.ds(h * CPH, CPH)], dst_half)
    pltpu.sync_copy(w_hbm.at[pl.ds((wid * NCHUNK + h * CPH) * CH, CPH * CH)],
                    w_half)
    pltpu.async_copy(x_hbm.at[src_half.at[0]], rows0, gsem0)

    def pair_body(p, carry):
      i0 = 2 * p
      pltpu.async_copy(x_hbm.at[src_half.at[i0 + 1]], rows1, gsem1)
      pltpu.make_async_copy(x_hbm.at[src_half.at[i0]], rows0, gsem0).wait()
      _scale_rows(rows0, w_half, i0)
      pltpu.sync_copy(rows0, acc.at[dst_half.at[i0]], add=True)

      @pl.when(p < CPH // 2 - 1)
      def _():
        pltpu.async_copy(x_hbm.at[src_half.at[i0 + 2]], rows0, gsem0)

      pltpu.make_async_copy(x_hbm.at[src_half.at[i0 + 1]], rows1, gsem1).wait()
      _scale_rows(rows1, w_half, i0 + 1)
      pltpu.sync_copy(rows1, acc.at[dst_half.at[i0 + 1]], add=True)
      return carry

    lax.fori_loop(0, CPH // 2, pair_body, 0)

  plsc.subcore_barrier()

  pltpu.sync_copy(acc.at[pl.ds(rbase, RPT)], out_hbm.at[c, pl.ds(rbase, RPT)])

  @pl.when(s == NS - 1)
  def _():
    pltpu.sync_copy(acc.at[pl.ds(NS * RPT, REM)],
                    out_hbm.at[c, pl.ds(NS * RPT, REM)])


@jax.jit
def _spmm_sc(x, src, dst, w, zeros):
  mesh = plsc.VectorSubcoreMesh(core_axis_name="c", subcore_axis_name="s")
  return pl.kernel(
      _spmm_body,
      out_type=jax.ShapeDtypeStruct((NC, N, F), jnp.float32),
      mesh=mesh,
      scratch_types=[
          pltpu.VMEM((CPH, CH), jnp.int32),
          pltpu.VMEM((CPH, CH), jnp.int32),
          pltpu.VMEM((CPH * CH,), jnp.float32),
          pltpu.VMEM((CH, F), jnp.float32),
          pltpu.VMEM((CH, F), jnp.float32),
          pltpu.VMEM_SHARED((N, F), jnp.float32),
          pltpu.SemaphoreType.DMA,
          pltpu.SemaphoreType.DMA,
      ],
  )(x, src, dst, w, zeros)


def _dense_body(x_ref, nb0_ref, nb1_ref, norm_ref, sel_ref,
                g1_ref, g2_ref, b1_ref, b2_ref, r_ref, w_ref,
                hk_ref, m_ref):
  x = x_ref[...]
  nb = nb0_ref[...] + nb1_ref[...]
  ga = jnp.dot(x, g1_ref[...], preferred_element_type=jnp.float32)
  ga += jnp.dot(nb, g2_ref[...], preferred_element_type=jnp.float32)
  gamma = jnp.where(ga >= 0, ga, 0.2 * ga) + 1.0
  be = jnp.dot(x, b1_ref[...], preferred_element_type=jnp.float32)
  be += jnp.dot(nb, b2_ref[...], preferred_element_type=jnp.float32)
  beta = jnp.where(be >= 0, be, 0.2 * be)
  m = x + gamma * r_ref[...] + beta - nb
  m_ref[...] = m
  t = x + nb + sel_ref[0, 0] * m
  hk = jnp.dot(t, w_ref[...], preferred_element_type=jnp.float32)
  hk_ref[...] = hk / (norm_ref[...] + 1.0)


BLK = 1000


@jax.jit
def _dense_tc(x, nb0, nb1, norm, sel, G1, G2, B1, B2, r, W):
  grid = (N // BLK,)
  row_spec = pl.BlockSpec((BLK, F), lambda i: (i, 0))
  full_spec = pl.BlockSpec((F, F), lambda i: (0, 0))
  return pl.pallas_call(
      _dense_body,
      grid=grid,
      in_specs=[
          row_spec, row_spec, row_spec,
          pl.BlockSpec((BLK, 1), lambda i: (i, 0)),
          pl.BlockSpec(memory_space=pltpu.SMEM),
          full_spec, full_spec, full_spec, full_spec,
          pl.BlockSpec((1, F), lambda i: (0, 0)),
          full_spec,
      ],
      out_specs=[row_spec, row_spec],
      out_shape=[
          jax.ShapeDtypeStruct((N, F), jnp.float32),
          jax.ShapeDtypeStruct((N, F), jnp.float32),
      ],
  )(x, nb0, nb1, norm, sel, G1, G2, B1, B2, r, W)


def kernel(x, edge_index, edge_weight, norm, head, G1, G2, B1, B2, r, W):
  src = edge_index[0]
  dst = edge_index[1]
  pad = E_PAD - E
  src_p = jnp.concatenate([src, jnp.zeros((pad,), jnp.int32)])
  src_p = src_p.reshape(NW, NCHUNK, CH)
  dst_p = jnp.concatenate([dst, jnp.zeros((pad,), jnp.int32)])
  dst_p = dst_p.reshape(NW, NCHUNK, CH)
  w_p = jnp.concatenate([edge_weight, jnp.zeros((pad,), jnp.float32)])
  zeros = jnp.zeros((N, F), jnp.float32)
  partial = _spmm_sc(x, src_p, dst_p, w_p, zeros)
  sel = jnp.where(jnp.asarray(head) != 0, 0.0, 1.0).astype(jnp.float32)
  sel = jnp.reshape(sel, (1, 1))
  hk, m = _dense_tc(x, partial[0], partial[1], norm, sel, G1, G2, B1, B2, r, W)
  return (hk, m)


# speedup vs baseline: 19.2794x; 3.0929x over previous
import jax
import jax.numpy as jnp
from jax import lax
from jax.experimental import pallas as pl
from jax.experimental.pallas import tpu as pltpu
import jax.experimental.pallas.tpu_sc as plsc

N = 10000
E = 320000
F = 128

NC = 2
NS = 16
NW = NC * NS

CH = 128
EPW = E // NW
NCHUNK = EPW // CH
TAIL = EPW - NCHUNK * CH
RPT = 624
REM = N - NS * RPT

_DNUMS = lax.GatherDimensionNumbers(
    offset_dims=(), collapsed_slice_dims=(0,), start_index_map=(0,))


def _bcast_lane(vec, e):
  idx = jnp.full((16, 1), e, jnp.int32)
  return lax.gather(vec, idx, _DNUMS, (1,),
                    mode=lax.GatherScatterMode.PROMISE_IN_BOUNDS)


def _scale_rows(rows_v, wrow, nedge):
  def group_body(g, carry):
    wv = wrow[pl.ds(g * 16, 16)]
    for e in range(16):
      wb = _bcast_lane(wv, e)
      row = g * 16 + e
      for j in range(F // 16):
        sl = pl.ds(j * 16, 16)
        rows_v[row, sl] = rows_v[row, sl] * wb
    return carry

  lax.fori_loop(0, nedge // 16, group_body, 0)


def _spmm_body(x_hbm, ei_hbm, w_hbm, out_hbm,
               srcv, dstv, wv, rows, msrc, mdst, mw, acc,
               isem0, isem1, isem2, gsem0, gsem1, gsem2):
  c = lax.axis_index("c")
  s = lax.axis_index("s")
  wid = c * NS + s
  ebase = wid * EPW
  rbase = s * RPT
  isems = (isem0, isem1, isem2)
  gsems = (gsem0, gsem1, gsem2)

  def idx_load(i, q):
    off = ebase + i * CH
    pltpu.async_copy(ei_hbm.at[pl.ds(off, CH)], srcv.at[q], isems[q])
    pltpu.async_copy(ei_hbm.at[pl.ds(E + off, CH)], dstv.at[q], isems[q])
    pltpu.async_copy(w_hbm.at[pl.ds(off, CH)], wv.at[q], isems[q])

  def idx_wait(i, q):
    off = ebase + i * CH
    pltpu.make_async_copy(ei_hbm.at[pl.ds(off, CH)], srcv.at[q],
                          isems[q]).wait()
    pltpu.make_async_copy(ei_hbm.at[pl.ds(E + off, CH)], dstv.at[q],
                          isems[q]).wait()
    pltpu.make_async_copy(w_hbm.at[pl.ds(off, CH)], wv.at[q],
                          isems[q]).wait()

  def gather_start(q):
    pltpu.async_copy(x_hbm.at[srcv.at[q]], rows.at[q], gsems[q])

  def gather_wait(q):
    pltpu.make_async_copy(x_hbm.at[srcv.at[q]], rows.at[q], gsems[q]).wait()

  idx_load(0, 0)
  idx_load(1, 1)
  idx_load(2, 2)

  def zfill(rr, carry):
    zero = jnp.zeros((16,), jnp.float32)
    for j in range(F // 16):
      rows[2, rr, pl.ds(j * 16, 16)] = zero
    return carry

  lax.fori_loop(0, CH, zfill, 0)
  for k in range(4):
    pltpu.sync_copy(rows.at[2], acc.at[pl.ds(rbase + k * CH, CH)])
  pltpu.sync_copy(rows.at[2, pl.ds(0, RPT - 4 * CH)],
                  acc.at[pl.ds(rbase + 4 * CH, RPT - 4 * CH)])

  @pl.when(s == NS - 1)
  def _():
    pltpu.sync_copy(rows.at[2, pl.ds(0, REM)],
                    acc.at[pl.ds(NS * RPT, REM)])

  idx_wait(0, 0)
  gather_start(0)
  idx_wait(1, 1)
  gather_start(1)

  plsc.subcore_barrier()

  def chunk_step(i, k, load_next, gather_next):
    gather_wait(k)
    _scale_rows(rows.at[k], wv.at[k], CH)
    pltpu.sync_copy(rows.at[k], acc.at[dstv.at[k]], add=True)
    if load_next:
      idx_load(i + 3, k)
    if gather_next:
      q2 = (k + 2) % 3
      idx_wait(i + 2, q2)
      gather_start(q2)

  def loop_body(g, carry):
    i0 = 3 * g
    for k in range(3):
      chunk_step(i0 + k, k, True, True)
    return carry

  lax.fori_loop(0, NCHUNK // 3 - 1, loop_body, 0)
  i0 = NCHUNK - 3
  chunk_step(i0, 0, False, True)
  chunk_step(i0 + 1, 1, False, False)
  chunk_step(i0 + 2, 2, False, False)

  toff = ebase + NCHUNK * CH
  pltpu.sync_copy(ei_hbm.at[pl.ds(toff, TAIL)], msrc)
  pltpu.sync_copy(ei_hbm.at[pl.ds(E + toff, TAIL)], mdst)
  pltpu.sync_copy(w_hbm.at[pl.ds(toff, TAIL)], mw)
  mrows = rows.at[0, pl.ds(0, TAIL)]
  pltpu.async_copy(x_hbm.at[msrc], mrows, gsem0).wait()
  wv16 = mw[...]
  for e in range(TAIL):
    wb = _bcast_lane(wv16, e)
    for j in range(F // 16):
      sl = pl.ds(j * 16, 16)
      rows[0, e, sl] = rows[0, e, sl] * wb
  pltpu.sync_copy(mrows, acc.at[mdst], add=True)

  plsc.subcore_barrier()

  pltpu.sync_copy(acc.at[pl.ds(rbase, RPT)], out_hbm.at[c, pl.ds(rbase, RPT)])

  @pl.when(s == NS - 1)
  def _():
    pltpu.sync_copy(acc.at[pl.ds(NS * RPT, REM)],
                    out_hbm.at[c, pl.ds(NS * RPT, REM)])


@jax.jit
def _spmm_sc(x, edge_index, edge_weight):
  mesh = plsc.VectorSubcoreMesh(core_axis_name="c", subcore_axis_name="s")
  return pl.kernel(
      _spmm_body,
      out_type=jax.ShapeDtypeStruct((NC, N, F), jnp.float32),
      mesh=mesh,
      scratch_types=[
          pltpu.VMEM((3, CH), jnp.int32),
          pltpu.VMEM((3, CH), jnp.int32),
          pltpu.VMEM((3, CH), jnp.float32),
          pltpu.VMEM((3, CH, F), jnp.float32),
          pltpu.VMEM((TAIL,), jnp.int32),
          pltpu.VMEM((TAIL,), jnp.int32),
          pltpu.VMEM((TAIL,), jnp.float32),
          pltpu.VMEM_SHARED((N, F), jnp.float32),
          pltpu.SemaphoreType.DMA,
          pltpu.SemaphoreType.DMA,
          pltpu.SemaphoreType.DMA,
          pltpu.SemaphoreType.DMA,
          pltpu.SemaphoreType.DMA,
          pltpu.SemaphoreType.DMA,
      ],
  )(x, edge_index, edge_weight)


def _dense_body(x_ref, nb0_ref, nb1_ref, norm_ref, sel_ref,
                g1_ref, g2_ref, b1_ref, b2_ref, r_ref, w_ref,
                hk_ref, m_ref):
  x = x_ref[...]
  nb = nb0_ref[...] + nb1_ref[...]
  ga = jnp.dot(x, g1_ref[...], preferred_element_type=jnp.float32)
  ga += jnp.dot(nb, g2_ref[...], preferred_element_type=jnp.float32)
  gamma = jnp.where(ga >= 0, ga, 0.2 * ga) + 1.0
  be = jnp.dot(x, b1_ref[...], preferred_element_type=jnp.float32)
  be += jnp.dot(nb, b2_ref[...], preferred_element_type=jnp.float32)
  beta = jnp.where(be >= 0, be, 0.2 * be)
  m = x + gamma * r_ref[...] + beta - nb
  m_ref[...] = m
  t = x + nb + sel_ref[0, 0] * m
  hk = jnp.dot(t, w_ref[...], preferred_element_type=jnp.float32)
  hk_ref[...] = hk / (norm_ref[...] + 1.0)


BLK = 1000


@jax.jit
def _dense_tc(x, nb0, nb1, norm, sel, G1, G2, B1, B2, r, W):
  grid = (N // BLK,)
  row_spec = pl.BlockSpec((BLK, F), lambda i: (i, 0))
  full_spec = pl.BlockSpec((F, F), lambda i: (0, 0))
  return pl.pallas_call(
      _dense_body,
      grid=grid,
      in_specs=[
          row_spec, row_spec, row_spec,
          pl.BlockSpec((BLK, 1), lambda i: (i, 0)),
          pl.BlockSpec(memory_space=pltpu.SMEM),
          full_spec, full_spec, full_spec, full_spec,
          pl.BlockSpec((1, F), lambda i: (0, 0)),
          full_spec,
      ],
      out_specs=[row_spec, row_spec],
      out_shape=[
          jax.ShapeDtypeStruct((N, F), jnp.float32),
          jax.ShapeDtypeStruct((N, F), jnp.float32),
      ],
  )(x, nb0, nb1, norm, sel, G1, G2, B1, B2, r, W)


def kernel(x, edge_index, edge_weight, norm, head, G1, G2, B1, B2, r, W):
  partial = _spmm_sc(x, jnp.reshape(edge_index, (2 * E,)), edge_weight)
  sel = jnp.where(jnp.asarray(head) != 0, 0.0, 1.0).astype(jnp.float32)
  sel = jnp.reshape(sel, (1, 1))
  hk, m = _dense_tc(x, partial[0], partial[1], norm, sel, G1, G2, B1, B2, r, W)
  return (hk, m)


# speedup vs baseline: 20.3750x; 1.0568x over previous
import jax
import jax.numpy as jnp
from jax import lax
from jax.experimental import pallas as pl
from jax.experimental.pallas import tpu as pltpu
import jax.experimental.pallas.tpu_sc as plsc

N = 10000
E = 320000
F = 128

NC = 2
NS = 16
NW = NC * NS

CH = 96
EPW = E // NW
NCHUNK = EPW // CH
TAIL = EPW - NCHUNK * CH
NB = 4
RPT = 624
REM = N - NS * RPT

_DNUMS = lax.GatherDimensionNumbers(
    offset_dims=(), collapsed_slice_dims=(0,), start_index_map=(0,))


def _bcast_lane(vec, e):
  idx = jnp.full((16, 1), e, jnp.int32)
  return lax.gather(vec, idx, _DNUMS, (1,),
                    mode=lax.GatherScatterMode.PROMISE_IN_BOUNDS)


def _scale_rows(rows_v, wrow, nedge):
  def group_body(g, carry):
    wv = wrow[pl.ds(g * 16, 16)]
    for e in range(16):
      wb = _bcast_lane(wv, e)
      row = g * 16 + e
      for j in range(F // 16):
        sl = pl.ds(j * 16, 16)
        rows_v[row, sl] = rows_v[row, sl] * wb
    return carry

  lax.fori_loop(0, nedge // 16, group_body, 0)


def _spmm_body(x_hbm, ei_hbm, w_hbm, out_hbm,
               srcv, dstv, wv, rows, msrc, mdst, mw, acc,
               isem0, isem1, isem2, isem3,
               gsem0, gsem1, gsem2, gsem3,
               ssem0, ssem1, ssem2, ssem3):
  c = lax.axis_index("c")
  s = lax.axis_index("s")
  wid = c * NS + s
  ebase = wid * EPW
  rbase = s * RPT
  isems = (isem0, isem1, isem2, isem3)
  gsems = (gsem0, gsem1, gsem2, gsem3)
  ssems = (ssem0, ssem1, ssem2, ssem3)

  def idx_load(i, q):
    off = ebase + i * CH
    pltpu.async_copy(ei_hbm.at[pl.ds(off, CH)], srcv.at[q], isems[q])
    pltpu.async_copy(ei_hbm.at[pl.ds(E + off, CH)], dstv.at[q], isems[q])
    pltpu.async_copy(w_hbm.at[pl.ds(off, CH)], wv.at[q], isems[q])

  def idx_wait(i, q):
    off = ebase + i * CH
    pltpu.make_async_copy(ei_hbm.at[pl.ds(off, CH)], srcv.at[q],
                          isems[q]).wait()
    pltpu.make_async_copy(ei_hbm.at[pl.ds(E + off, CH)], dstv.at[q],
                          isems[q]).wait()
    pltpu.make_async_copy(w_hbm.at[pl.ds(off, CH)], wv.at[q],
                          isems[q]).wait()

  def gather_start(q):
    pltpu.async_copy(x_hbm.at[srcv.at[q]], rows.at[q], gsems[q])

  def gather_wait(q):
    pltpu.make_async_copy(x_hbm.at[srcv.at[q]], rows.at[q], gsems[q]).wait()

  def scatter_start(q):
    pltpu.async_copy(rows.at[q], acc.at[dstv.at[q]], ssems[q], add=True)

  def scatter_wait(q):
    pltpu.make_async_copy(rows.at[q], acc.at[dstv.at[q]], ssems[q]).wait()

  idx_load(0, 0)
  idx_load(1, 1)
  idx_load(2, 2)

  def zfill(rr, carry):
    zero = jnp.zeros((16,), jnp.float32)
    for j in range(F // 16):
      rows[3, rr, pl.ds(j * 16, 16)] = zero
    return carry

  lax.fori_loop(0, CH, zfill, 0)
  for k in range(RPT // CH):
    pltpu.sync_copy(rows.at[3], acc.at[pl.ds(rbase + k * CH, CH)])
  lastoff = (RPT // CH) * CH
  pltpu.sync_copy(rows.at[3, pl.ds(0, RPT - lastoff)],
                  acc.at[pl.ds(rbase + lastoff, RPT - lastoff)])

  @pl.when(s == NS - 1)
  def _():
    pltpu.sync_copy(rows.at[3, pl.ds(0, REM)],
                    acc.at[pl.ds(NS * RPT, REM)])

  idx_wait(0, 0)
  gather_start(0)
  idx_wait(1, 1)
  gather_start(1)

  plsc.subcore_barrier()

  def chunk_step(i, k, wait_prev, load_next, gather_next):
    gather_wait(k)
    _scale_rows(rows.at[k], wv.at[k], CH)
    if wait_prev:
      scatter_wait((k + 3) % NB)
    scatter_start(k)
    if load_next:
      idx_load(i + 3, (k + 3) % NB)
    if gather_next:
      q2 = (k + 2) % NB
      idx_wait(i + 2, q2)
      gather_start(q2)

  chunk_step(0, 0, False, True, True)
  chunk_step(1, 1, True, True, True)
  chunk_step(2, 2, True, True, True)
  chunk_step(3, 3, True, True, True)

  def loop_body(g, carry):
    i0 = 4 + 4 * g
    for k in range(NB):
      chunk_step(i0 + k, k, True, True, True)
    return carry

  lax.fori_loop(0, (NCHUNK - 8) // NB, loop_body, 0)

  i0 = NCHUNK - 4
  chunk_step(i0, 0, True, True, True)
  chunk_step(i0 + 1, 1, True, False, True)
  chunk_step(i0 + 2, 2, True, False, False)
  chunk_step(i0 + 3, 3, True, False, False)
  scatter_wait(3)

  toff = ebase + NCHUNK * CH
  pltpu.sync_copy(ei_hbm.at[pl.ds(toff, TAIL)], msrc)
  pltpu.sync_copy(ei_hbm.at[pl.ds(E + toff, TAIL)], mdst)
  pltpu.sync_copy(w_hbm.at[pl.ds(toff, TAIL)], mw)
  mrows = rows.at[0, pl.ds(0, TAIL)]
  pltpu.async_copy(x_hbm.at[msrc], mrows, gsem0).wait()
  wv16 = mw[...]
  for e in range(TAIL):
    wb = _bcast_lane(wv16, e)
    for j in range(F // 16):
      sl = pl.ds(j * 16, 16)
      rows[0, e, sl] = rows[0, e, sl] * wb
  pltpu.sync_copy(mrows, acc.at[mdst], add=True)

  plsc.subcore_barrier()

  pltpu.sync_copy(acc.at[pl.ds(rbase, RPT)], out_hbm.at[c, pl.ds(rbase, RPT)])

  @pl.when(s == NS - 1)
  def _():
    pltpu.sync_copy(acc.at[pl.ds(NS * RPT, REM)],
                    out_hbm.at[c, pl.ds(NS * RPT, REM)])


@jax.jit
def _spmm_sc(x, edge_index, edge_weight):
  mesh = plsc.VectorSubcoreMesh(core_axis_name="c", subcore_axis_name="s")
  return pl.kernel(
      _spmm_body,
      out_type=jax.ShapeDtypeStruct((NC, N, F), jnp.float32),
      mesh=mesh,
      scratch_types=[
          pltpu.VMEM((NB, CH), jnp.int32),
          pltpu.VMEM((NB, CH), jnp.int32),
          pltpu.VMEM((NB, CH), jnp.float32),
          pltpu.VMEM((NB, CH, F), jnp.float32),
          pltpu.VMEM((TAIL,), jnp.int32),
          pltpu.VMEM((TAIL,), jnp.int32),
          pltpu.VMEM((TAIL,), jnp.float32),
          pltpu.VMEM_SHARED((N, F), jnp.float32),
          pltpu.SemaphoreType.DMA,
          pltpu.SemaphoreType.DMA,
          pltpu.SemaphoreType.DMA,
          pltpu.SemaphoreType.DMA,
          pltpu.SemaphoreType.DMA,
          pltpu.SemaphoreType.DMA,
          pltpu.SemaphoreType.DMA,
          pltpu.SemaphoreType.DMA,
          pltpu.SemaphoreType.DMA,
          pltpu.SemaphoreType.DMA,
          pltpu.SemaphoreType.DMA,
          pltpu.SemaphoreType.DMA,
      ],
  )(x, edge_index, edge_weight)


def _dense_body(x_ref, nb0_ref, nb1_ref, norm_ref, head_ref,
                g1_ref, g2_ref, b1_ref, b2_ref, r_ref, w_ref,
                hk_ref, m_ref):
  x = x_ref[...]
  nb = nb0_ref[...] + nb1_ref[...]
  ga = jnp.dot(x, g1_ref[...], preferred_element_type=jnp.float32)
  ga += jnp.dot(nb, g2_ref[...], preferred_element_type=jnp.float32)
  gamma = jnp.where(ga >= 0, ga, 0.2 * ga) + 1.0
  be = jnp.dot(x, b1_ref[...], preferred_element_type=jnp.float32)
  be += jnp.dot(nb, b2_ref[...], preferred_element_type=jnp.float32)
  beta = jnp.where(be >= 0, be, 0.2 * be)
  m = x + gamma * r_ref[...] + beta - nb
  m_ref[...] = m
  sel = jnp.where(head_ref[0, 0] != 0, 0.0, 1.0)
  t = x + nb + sel * m
  hk = jnp.dot(t, w_ref[...], preferred_element_type=jnp.float32)
  hk_ref[...] = hk / (norm_ref[...] + 1.0)


BLK = 1000


@jax.jit
def _dense_tc(x, nb0, nb1, norm, head, G1, G2, B1, B2, r, W):
  grid = (N // BLK,)
  row_spec = pl.BlockSpec((BLK, F), lambda i: (i, 0))
  full_spec = pl.BlockSpec((F, F), lambda i: (0, 0))
  return pl.pallas_call(
      _dense_body,
      grid=grid,
      in_specs=[
          row_spec, row_spec, row_spec,
          pl.BlockSpec((BLK, 1), lambda i: (i, 0)),
          pl.BlockSpec(memory_space=pltpu.SMEM),
          full_spec, full_spec, full_spec, full_spec,
          pl.BlockSpec((1, F), lambda i: (0, 0)),
          full_spec,
      ],
      out_specs=[row_spec, row_spec],
      out_shape=[
          jax.ShapeDtypeStruct((N, F), jnp.float32),
          jax.ShapeDtypeStruct((N, F), jnp.float32),
      ],
  )(x, nb0, nb1, norm, head, G1, G2, B1, B2, r, W)


def kernel(x, edge_index, edge_weight, norm, head, G1, G2, B1, B2, r, W):
  partial = _spmm_sc(x, jnp.reshape(edge_index, (2 * E,)), edge_weight)
  head_arr = jnp.reshape(jnp.asarray(head, jnp.int32), (1, 1))
  hk, m = _dense_tc(x, partial[0], partial[1], norm, head_arr,
                    G1, G2, B1, B2, r, W)
  return (hk, m)
